# speedup vs baseline: 1.2574x; 1.2574x over previous
import functools

import jax
import jax.numpy as jnp
from jax import lax
from jax.experimental import pallas as pl
from jax.experimental.pallas import tpu as pltpu

AM_MARGIN = 0.2
AM_SCALE = 30.0
PROTO_W = 10.0
PROTO_B = -5.0
MTL_WEIGHT = 0.6


def _l2n_bf16(v):
    s = jnp.sum(v * v, axis=-1, keepdims=True)
    return (v * lax.rsqrt(jnp.maximum(s, 1e-24))).astype(jnp.bfloat16)


def _lse_tgt_max(logits, mask):
    m = jnp.max(logits, axis=-1, keepdims=True)
    se = jnp.sum(jnp.exp(logits - m), axis=-1, keepdims=True)
    lse = m + jnp.log(se)
    tgt = jnp.sum(jnp.where(mask, logits, 0.0), axis=-1, keepdims=True)
    return lse, tgt, m


def _fused_kernel(x_ref, labr_ref, pos_ref, anc_ref, labh_ref,
                  w_am_hbm, w_px_hbm,
                  out_ref,
                  wf32, wn_am, wn_px, pn_ref, an_ref,
                  sem_am, sem_px,
                  *, rows_am, rows_m, b_full, c, ch):
    f32 = jnp.float32
    s = pl.program_id(0)

    cp_am = pltpu.make_async_copy(w_am_hbm, wf32, sem_am)
    cp_am.start()

    pn_ref[...] = _l2n_bf16(pos_ref[...])
    an_ref[...] = _l2n_bf16(anc_ref[...])

    cp_am.wait()
    w = wf32[...]
    inv = lax.rsqrt(jnp.maximum(jnp.sum(w * w, axis=0, keepdims=True), 1e-24))
    wn_am[...] = (w * inv).astype(jnp.bfloat16)

    cp_px = pltpu.make_async_copy(w_px_hbm, wf32, sem_px)
    cp_px.start()

    sm = AM_SCALE * AM_MARGIN
    ce_sum = jnp.zeros((), f32)
    hits = jnp.zeros((), f32)
    for t in range(rows_am // ch):
        xn = _l2n_bf16(x_ref[t * ch:(t + 1) * ch, :])
        cos = jnp.dot(xn, wn_am[...], preferred_element_type=f32)
        lab = labr_ref[t * ch:(t + 1) * ch, :]
        cls = lax.broadcasted_iota(jnp.int32, (ch, c), 1)
        mask = cls == lab
        scaled = AM_SCALE * cos
        logits = jnp.where(mask, scaled - sm, scaled)
        lse, tgt, m = _lse_tgt_max(logits, mask)
        ce_sum = ce_sum + jnp.sum(lse - tgt)
        hits = hits + jnp.sum(jnp.where(tgt >= m, 1.0, 0.0))

    pair = PROTO_W * lax.dot_general(
        pn_ref[...], an_ref[...], (((1,), (1,)), ((), ())),
        preferred_element_type=f32) + PROTO_B
    row0 = s * rows_m
    ri = lax.broadcasted_iota(jnp.int32, (rows_m, b_full), 0) + row0
    ci = lax.broadcasted_iota(jnp.int32, (rows_m, b_full), 1)
    lse_p, tgt_p, _ = _lse_tgt_max(pair, ri == ci)
    pair_sum = jnp.sum(lse_p - tgt_p)

    cp_px.wait()
    w2 = wf32[...]
    inv2 = lax.rsqrt(jnp.maximum(jnp.sum(w2 * w2, axis=0, keepdims=True), 1e-24))
    wn_px[...] = (w2 * inv2).astype(jnp.bfloat16)

    prox = PROTO_W * jnp.dot(pn_ref[...], wn_px[...],
                             preferred_element_type=f32) + PROTO_B
    clsx = lax.broadcasted_iota(jnp.int32, (rows_m, c), 1)
    lse_x, tgt_x, _ = _lse_tgt_max(prox, clsx == labh_ref[...])
    proxy_sum = jnp.sum(lse_x - tgt_x)

    sub = lax.broadcasted_iota(jnp.int32, (8, 128), 0)
    lane = lax.broadcasted_iota(jnp.int32, (8, 128), 1)
    r0 = sub == 0
    out_ref[0] = jnp.where(r0 & (lane == 0), ce_sum,
                 jnp.where(r0 & (lane == 1), hits,
                 jnp.where(r0 & (lane == 2), pair_sum,
                 jnp.where(r0 & (lane == 3), proxy_sum, 0.0))))


def kernel(x, label, w_am, w_proxy):
    B, M, D = x.shape
    C = w_am.shape[1]
    assert M >= 2
    assert D % 128 == 0 and C % 128 == 0 and B % 2 == 0
    N = B * M
    assert N % 2 == 0

    S = 2
    ROWS_AM = N // S
    ROWS_M = B // S
    CH = 256 if ROWS_AM % 256 == 0 else ROWS_AM

    f32 = jnp.float32
    x = x.astype(f32)
    x_flat = x.reshape(N, D)
    lab_rep = jnp.repeat(label.astype(jnp.int32), M).reshape(N, 1)
    positive = x[:, 0, :]
    anchor = jnp.mean(x[:, 1:, :], axis=1)
    spk = label.astype(jnp.int32).reshape(B, 1)

    cost = pl.CostEstimate(
        flops=2 * N * D * C + 2 * B * D * C + 2 * B * B * D,
        transcendentals=N * C + B * C + B * B,
        bytes_accessed=2 * D * C * 4 + N * D * 4 + 2 * B * D * 4 + S * 8 * 128 * 4)

    fused = functools.partial(_fused_kernel, rows_am=ROWS_AM, rows_m=ROWS_M,
                              b_full=B, c=C, ch=CH)
    parts = pl.pallas_call(
        fused,
        out_shape=jax.ShapeDtypeStruct((S, 8, 128), f32),
        grid=(S,),
        in_specs=[
            pl.BlockSpec((ROWS_AM, D), lambda s: (s, 0)),
            pl.BlockSpec((ROWS_AM, 1), lambda s: (s, 0)),
            pl.BlockSpec((ROWS_M, D), lambda s: (s, 0)),
            pl.BlockSpec((B, D), lambda s: (0, 0)),
            pl.BlockSpec((ROWS_M, 1), lambda s: (s, 0)),
            pl.BlockSpec(memory_space=pl.ANY),
            pl.BlockSpec(memory_space=pl.ANY),
        ],
        out_specs=pl.BlockSpec((1, 8, 128), lambda s: (s, 0, 0)),
        scratch_shapes=[
            pltpu.VMEM((D, C), f32),
            pltpu.VMEM((D, C), jnp.bfloat16),
            pltpu.VMEM((D, C), jnp.bfloat16),
            pltpu.VMEM((ROWS_M, D), jnp.bfloat16),
            pltpu.VMEM((B, D), jnp.bfloat16),
            pltpu.SemaphoreType.DMA,
            pltpu.SemaphoreType.DMA,
        ],
        compiler_params=pltpu.CompilerParams(
            dimension_semantics=("parallel",),
            vmem_limit_bytes=56 * 1024 * 1024),
        cost_estimate=cost,
    )(x_flat, lab_rep, positive, anchor, spk, w_am.astype(f32),
      w_proxy.astype(f32))

    ce_sum = jnp.sum(parts[:, 0, 0])
    hits = jnp.sum(parts[:, 0, 1])
    pair_total = jnp.sum(parts[:, 0, 2])
    proxy_total = jnp.sum(parts[:, 0, 3])

    loss_ce = ce_sum / float(N)
    prec1 = 100.0 * hits / float(N)
    loss_ml = 0.5 * (pair_total / float(B)) + 0.5 * (proxy_total / float(B))
    loss = (1.0 - MTL_WEIGHT) * loss_ce + MTL_WEIGHT * loss_ml
    return loss, prec1


# speedup vs baseline: 1.5968x; 1.2698x over previous
import functools
import math

import jax
import jax.numpy as jnp
from jax import lax
from jax.experimental import pallas as pl
from jax.experimental.pallas import tpu as pltpu

AM_MARGIN = 0.2
AM_SCALE = 30.0
PROTO_W = 10.0
PROTO_B = -5.0
MTL_WEIGHT = 0.6

_SM = AM_SCALE * AM_MARGIN
_EXP_NEG_SM = math.exp(-_SM)
_NEG = -1e30


def _l2n_bf16(v):
    s = jnp.sum(v * v, axis=-1, keepdims=True)
    return (v * lax.rsqrt(jnp.maximum(s, 1e-24))).astype(jnp.bfloat16)


def _colnorm_bf16(w):
    inv = lax.rsqrt(jnp.maximum(jnp.sum(w * w, axis=0, keepdims=True), 1e-24))
    return (w * inv).astype(jnp.bfloat16)


def _main_kernel(x_ref, labr_ref, pos_ref, anc_ref, labh_ref,
                 w_am_hbm, w_px_hbm,
                 out_ref,
                 wamf, wpxf, wn_am, wn_px, pn_ref, an_ref,
                 sem_am, sem_px,
                 *, n, b, hb, hc, ch):
    f32 = jnp.float32
    s = pl.program_id(0)
    col0 = pl.multiple_of(s * hc, 256)

    cp_am = pltpu.make_async_copy(w_am_hbm.at[:, pl.ds(col0, hc)], wamf, sem_am)
    cp_am.start()
    cp_px = pltpu.make_async_copy(w_px_hbm.at[:, pl.ds(col0, hc)], wpxf, sem_px)
    cp_px.start()

    pn_ref[...] = _l2n_bf16(pos_ref[...])
    an_ref[...] = _l2n_bf16(anc_ref[...])

    pair = PROTO_W * lax.dot_general(
        pn_ref[...], an_ref[...], (((1,), (1,)), ((), ())),
        preferred_element_type=f32) + PROTO_B
    acol0 = s * hb
    ri = lax.broadcasted_iota(jnp.int32, (b, hb), 0)
    ci = lax.broadcasted_iota(jnp.int32, (b, hb), 1) + acol0
    pmask = ri == ci
    ep = jnp.exp(pair)
    out_ref[0, 0:b, 3:4] = jnp.sum(ep, axis=-1, keepdims=True)
    out_ref[0, 0:b, 4:5] = jnp.sum(jnp.where(pmask, pair, 0.0), axis=-1,
                                   keepdims=True)

    cp_am.wait()
    wn_am[...] = _colnorm_bf16(wamf[...])

    cls = lax.broadcasted_iota(jnp.int32, (ch, hc), 1)
    for t in range(n // ch):
        r = slice(t * ch, (t + 1) * ch)
        xn = _l2n_bf16(x_ref[r, :])
        cos = jnp.dot(xn, wn_am[...], preferred_element_type=f32)
        scaled = AM_SCALE * cos
        mask = cls == (labr_ref[r, :] - col0)
        e = jnp.exp(scaled)
        se = jnp.sum(e * jnp.where(mask, _EXP_NEG_SM, 1.0), axis=-1,
                     keepdims=True)
        tgt = jnp.sum(jnp.where(mask, scaled, 0.0), axis=-1, keepdims=True)
        mx_non = jnp.max(jnp.where(mask, _NEG, scaled), axis=-1, keepdims=True)
        out_ref[0, r, 0:1] = se
        out_ref[0, r, 1:2] = tgt
        out_ref[0, r, 2:3] = mx_non

    cp_px.wait()
    wn_px[...] = _colnorm_bf16(wpxf[...])

    prox = PROTO_W * jnp.dot(pn_ref[...], wn_px[...],
                             preferred_element_type=f32) + PROTO_B
    clsx = lax.broadcasted_iota(jnp.int32, (b, hc), 1)
    xmask = clsx == (labh_ref[...] - col0)
    out_ref[0, 0:b, 5:6] = jnp.sum(jnp.exp(prox), axis=-1, keepdims=True)
    out_ref[0, 0:b, 6:7] = jnp.sum(jnp.where(xmask, prox, 0.0), axis=-1,
                                   keepdims=True)
    out_ref[0, 0:b, 7:8] = jnp.zeros((b, 1), f32)


def _combine_kernel(st_ref, out_ref, *, n, b):
    f32 = jnp.float32
    a0 = st_ref[0]
    a1 = st_ref[1]

    se = a0[:, 0:1] + a1[:, 0:1]
    tgt = (a0[:, 1:2] + a1[:, 1:2]) - _SM
    mx_non = jnp.maximum(a0[:, 2:3], a1[:, 2:3])
    lse = jnp.log(se)
    ce_sum = jnp.sum(lse - tgt)
    hits = jnp.sum(jnp.where(tgt >= mx_non, 1.0, 0.0))

    sep = a0[0:b, 3:4] + a1[0:b, 3:4]
    tgtp = a0[0:b, 4:5] + a1[0:b, 4:5]
    pair_sum = jnp.sum(jnp.log(sep) - tgtp)

    sex = a0[0:b, 5:6] + a1[0:b, 5:6]
    tgtx = a0[0:b, 6:7] + a1[0:b, 6:7]
    proxy_sum = jnp.sum(jnp.log(sex) - tgtx)

    loss_ce = ce_sum / float(n)
    prec1 = 100.0 * hits / float(n)
    loss_ml = 0.5 * (pair_sum / float(b)) + 0.5 * (proxy_sum / float(b))
    out_ref[0, 0] = (1.0 - MTL_WEIGHT) * loss_ce + MTL_WEIGHT * loss_ml
    out_ref[0, 1] = prec1


def kernel(x, label, w_am, w_proxy):
    B, M, D = x.shape
    C = w_am.shape[1]
    assert M >= 2
    N = B * M
    HC = C // 2
    HB = B // 2
    assert D % 128 == 0 and HC % 128 == 0 and HB % 8 == 0 and N % 8 == 0
    CH = 256 if N % 256 == 0 else N

    f32 = jnp.float32
    x = x.astype(f32)
    x_flat = x.reshape(N, D)
    lab_rep = jnp.repeat(label.astype(jnp.int32), M).reshape(N, 1)
    positive = x[:, 0, :]
    anchor = jnp.mean(x[:, 1:, :], axis=1)
    spk = label.astype(jnp.int32).reshape(B, 1)

    cost = pl.CostEstimate(
        flops=2 * N * D * C + 2 * B * D * C + 2 * B * B * D,
        transcendentals=N * C + B * C + B * B,
        bytes_accessed=2 * D * C * 4 + (N + 3 * B) * D * 4 + 2 * N * 8 * 4)

    main = functools.partial(_main_kernel, n=N, b=B, hb=HB, hc=HC, ch=CH)
    stats = pl.pallas_call(
        main,
        out_shape=jax.ShapeDtypeStruct((2, N, 8), f32),
        grid=(2,),
        in_specs=[
            pl.BlockSpec((N, D), lambda s: (0, 0)),
            pl.BlockSpec((N, 1), lambda s: (0, 0)),
            pl.BlockSpec((B, D), lambda s: (0, 0)),
            pl.BlockSpec((HB, D), lambda s: (s, 0)),
            pl.BlockSpec((B, 1), lambda s: (0, 0)),
            pl.BlockSpec(memory_space=pl.ANY),
            pl.BlockSpec(memory_space=pl.ANY),
        ],
        out_specs=pl.BlockSpec((1, N, 8), lambda s: (s, 0, 0)),
        scratch_shapes=[
            pltpu.VMEM((D, HC), f32),
            pltpu.VMEM((D, HC), f32),
            pltpu.VMEM((D, HC), jnp.bfloat16),
            pltpu.VMEM((D, HC), jnp.bfloat16),
            pltpu.VMEM((B, D), jnp.bfloat16),
            pltpu.VMEM((HB, D), jnp.bfloat16),
            pltpu.SemaphoreType.DMA,
            pltpu.SemaphoreType.DMA,
        ],
        compiler_params=pltpu.CompilerParams(
            dimension_semantics=("parallel",),
            vmem_limit_bytes=56 * 1024 * 1024),
        cost_estimate=cost,
    )(x_flat, lab_rep, positive, anchor, spk, w_am.astype(f32),
      w_proxy.astype(f32))

    comb = functools.partial(_combine_kernel, n=N, b=B)
    res = pl.pallas_call(
        comb,
        out_shape=jax.ShapeDtypeStruct((1, 2), f32),
        grid=(1,),
        in_specs=[pl.BlockSpec((2, N, 8), lambda i: (0, 0, 0))],
        out_specs=pl.BlockSpec(memory_space=pltpu.MemorySpace.SMEM),
        compiler_params=pltpu.CompilerParams(
            dimension_semantics=("arbitrary",)),
    )(stats)

    return res[0, 0], res[0, 1]


# speedup vs baseline: 2.0566x; 1.2880x over previous
import functools
import math

import jax
import jax.numpy as jnp
from jax import lax
from jax.experimental import pallas as pl
from jax.experimental.pallas import tpu as pltpu

AM_MARGIN = 0.2
AM_SCALE = 30.0
PROTO_W = 10.0
PROTO_B = -5.0
MTL_WEIGHT = 0.6

_SM = AM_SCALE * AM_MARGIN
_EXP_NEG_SM = math.exp(-_SM)
_LOG2E = 1.4426950408889634
_K_AM = AM_SCALE * _LOG2E
_K_PR = PROTO_W * _LOG2E
_NEG = -1e30


def _l2n_bf16(v):
    s = jnp.sum(v * v, axis=-1, keepdims=True)
    return (v * lax.rsqrt(jnp.maximum(s, 1e-24))).astype(jnp.bfloat16)


def _colnorm_bf16(w):
    inv = lax.rsqrt(jnp.maximum(jnp.sum(w * w, axis=0, keepdims=True), 1e-24))
    return (w * inv).astype(jnp.bfloat16)


def _main_kernel(x_ref, labr_ref, labh_ref,
                 w_am_hbm, w_px_hbm,
                 out_ref,
                 wamf, wpxf, wn_am, wn_px, pn_ref, an_ref,
                 sem_am, sem_px,
                 *, n, b, m_utts, hb, hc, ch):
    f32 = jnp.float32
    s = pl.program_id(0)
    col0 = pl.multiple_of(s * hc, 256)

    cp_am = pltpu.make_async_copy(w_am_hbm.at[:, pl.ds(col0, hc)], wamf, sem_am)
    cp_am.start()
    cp_px = pltpu.make_async_copy(w_px_hbm.at[:, pl.ds(col0, hc)], wpxf, sem_px)
    cp_px.start()

    pn_ref[...] = _l2n_bf16(x_ref[:, 0, :])
    arow0 = pl.multiple_of(s * hb, 8)
    anc = x_ref[pl.ds(arow0, hb), 1, :]
    for m in range(2, m_utts):
        anc = anc + x_ref[pl.ds(arow0, hb), m, :]
    if m_utts > 2:
        anc = anc * (1.0 / float(m_utts - 1))
    an_ref[...] = _l2n_bf16(anc)

    pairc = lax.dot_general(pn_ref[...], an_ref[...], (((1,), (1,)), ((), ())),
                            preferred_element_type=f32)
    ri = lax.broadcasted_iota(jnp.int32, (b, hb), 0)
    ci = lax.broadcasted_iota(jnp.int32, (b, hb), 1) + s * hb
    pmask = ri == ci
    out_ref[0, 0:b, 3:4] = jnp.sum(jnp.exp2(_K_PR * pairc), axis=-1,
                                   keepdims=True)
    out_ref[0, 0:b, 4:5] = jnp.sum(jnp.where(pmask, pairc, 0.0), axis=-1,
                                   keepdims=True)

    cp_am.wait()
    wn_am[...] = _colnorm_bf16(wamf[...])

    spc = ch // m_utts
    for t in range(n // ch):
        xn = _l2n_bf16(x_ref[t * spc:(t + 1) * spc, :, :].reshape(ch, -1))
        cos = jnp.dot(xn, wn_am[...], preferred_element_type=f32)
        cls = lax.broadcasted_iota(jnp.int32, (ch, hc), 1)
        mask = cls == (labr_ref[t * ch:(t + 1) * ch, :] - col0)
        r = slice(t * ch, (t + 1) * ch)
        out_ref[0, r, 0:1] = jnp.sum(jnp.exp2(_K_AM * cos), axis=-1,
                                     keepdims=True)
        out_ref[0, r, 1:2] = jnp.sum(jnp.where(mask, cos, 0.0), axis=-1,
                                     keepdims=True)
        out_ref[0, r, 2:3] = jnp.max(jnp.where(mask, _NEG, cos), axis=-1,
                                     keepdims=True)

    cp_px.wait()
    wn_px[...] = _colnorm_bf16(wpxf[...])

    pxc = jnp.dot(pn_ref[...], wn_px[...], preferred_element_type=f32)
    clsx = lax.broadcasted_iota(jnp.int32, (b, hc), 1)
    xmask = clsx == (labh_ref[...] - col0)
    out_ref[0, 0:b, 5:6] = jnp.sum(jnp.exp2(_K_PR * pxc), axis=-1,
                                   keepdims=True)
    out_ref[0, 0:b, 6:7] = jnp.sum(jnp.where(xmask, pxc, 0.0), axis=-1,
                                   keepdims=True)


def _combine_kernel(st_ref, out_ref, *, n, b):
    a0 = st_ref[0]
    a1 = st_ref[1]

    se_raw = a0[:, 0:1] + a1[:, 0:1]
    tc = a0[:, 1:2] + a1[:, 1:2]
    mxc = jnp.maximum(a0[:, 2:3], a1[:, 2:3])
    e_t = jnp.exp2(_K_AM * tc)
    se = se_raw + e_t * (_EXP_NEG_SM - 1.0)
    lse = jnp.log(se)
    tgt = AM_SCALE * tc - _SM
    ce_sum = jnp.sum(lse - tgt)
    hits = jnp.sum(jnp.where(tgt >= AM_SCALE * mxc, 1.0, 0.0))

    sp = a0[0:b, 3:4] + a1[0:b, 3:4]
    tp = a0[0:b, 4:5] + a1[0:b, 4:5]
    pair_sum = jnp.sum((jnp.log(sp) + PROTO_B) - (PROTO_W * tp + PROTO_B))

    sx = a0[0:b, 5:6] + a1[0:b, 5:6]
    tx = a0[0:b, 6:7] + a1[0:b, 6:7]
    proxy_sum = jnp.sum((jnp.log(sx) + PROTO_B) - (PROTO_W * tx + PROTO_B))

    loss_ce = ce_sum / float(n)
    prec1 = 100.0 * hits / float(n)
    loss_ml = 0.5 * (pair_sum / float(b)) + 0.5 * (proxy_sum / float(b))
    out_ref[0, 0] = (1.0 - MTL_WEIGHT) * loss_ce + MTL_WEIGHT * loss_ml
    out_ref[0, 1] = prec1


def kernel(x, label, w_am, w_proxy):
    B, M, D = x.shape
    C = w_am.shape[1]
    assert M >= 2
    N = B * M
    HC = C // 2
    HB = B // 2
    assert D % 128 == 0 and HC % 128 == 0 and HB % 8 == 0 and N % 8 == 0
    CH = 512 if (N % 512 == 0 and 512 % M == 0) else N
    assert CH % M == 0

    f32 = jnp.float32
    x = x.astype(f32)
    lab_rep = jnp.repeat(label.astype(jnp.int32), M).reshape(N, 1)
    spk = label.astype(jnp.int32).reshape(B, 1)

    cost = pl.CostEstimate(
        flops=2 * N * D * C + 2 * B * D * C + 2 * B * B * D,
        transcendentals=N * C + B * C + B * B,
        bytes_accessed=2 * D * C * 4 + N * D * 4 + 2 * N * 8 * 4)

    main = functools.partial(_main_kernel, n=N, b=B, m_utts=M, hb=HB, hc=HC,
                             ch=CH)
    stats = pl.pallas_call(
        main,
        out_shape=jax.ShapeDtypeStruct((2, N, 8), f32),
        grid=(2,),
        in_specs=[
            pl.BlockSpec((B, M, D), lambda s: (0, 0, 0)),
            pl.BlockSpec((N, 1), lambda s: (0, 0)),
            pl.BlockSpec((B, 1), lambda s: (0, 0)),
            pl.BlockSpec(memory_space=pl.ANY),
            pl.BlockSpec(memory_space=pl.ANY),
        ],
        out_specs=pl.BlockSpec((1, N, 8), lambda s: (s, 0, 0)),
        scratch_shapes=[
            pltpu.VMEM((D, HC), f32),
            pltpu.VMEM((D, HC), f32),
            pltpu.VMEM((D, HC), jnp.bfloat16),
            pltpu.VMEM((D, HC), jnp.bfloat16),
            pltpu.VMEM((B, D), jnp.bfloat16),
            pltpu.VMEM((HB, D), jnp.bfloat16),
            pltpu.SemaphoreType.DMA,
            pltpu.SemaphoreType.DMA,
        ],
        compiler_params=pltpu.CompilerParams(
            dimension_semantics=("parallel",),
            vmem_limit_bytes=56 * 1024 * 1024),
        cost_estimate=cost,
    )(x, lab_rep, spk, w_am.astype(f32), w_proxy.astype(f32))

    comb = functools.partial(_combine_kernel, n=N, b=B)
    res = pl.pallas_call(
        comb,
        out_shape=jax.ShapeDtypeStruct((1, 2), f32),
        grid=(1,),
        in_specs=[pl.BlockSpec((2, N, 8), lambda i: (0, 0, 0))],
        out_specs=pl.BlockSpec(memory_space=pltpu.MemorySpace.SMEM),
        compiler_params=pltpu.CompilerParams(
            dimension_semantics=("arbitrary",)),
    )(stats)

    return res[0, 0], res[0, 1]
